# baseline (speedup 1.0000x reference)
import functools

import jax
import jax.numpy as jnp
from jax import lax
from jax.experimental import pallas as pl
from jax.experimental.pallas import tpu as pltpu
from jax.experimental.pallas import tpu_sc as plsc

N = 64
OBS_DIM = 1024
LOWD = 512
K = 512
H = 8
A = 128
HID = 512

NW = 32
LN = 16
RPW = K * H // NW
CPW = K // NW

_sc_mesh = plsc.VectorSubcoreMesh(core_axis_name="c", subcore_axis_name="s")


@functools.partial(
    pl.kernel,
    out_type=jax.ShapeDtypeStruct((K, HID), jnp.float32),
    mesh=_sc_mesh,
    scratch_types=[
        pltpu.VMEM((RPW,), jnp.int32),
        pltpu.VMEM((RPW,), jnp.int32),
        pltpu.VMEM((RPW, HID), jnp.float32),
        pltpu.VMEM((CPW, HID), jnp.float32),
        pltpu.SemaphoreType.DMA,
    ],
)
def _sc_act_proj(w1_hbm, ai_hbm, out_hbm, idx_v, gidx_v, rows_v, acc_v, sem):
    wid = lax.axis_index("s") * 2 + lax.axis_index("c")
    base_r = wid * RPW
    base_k = wid * CPW
    pltpu.sync_copy(ai_hbm.at[pl.ds(base_r, RPW)], idx_v)
    for h in range(H):
        gidx_v[pl.ds(h * LN, LN)] = idx_v[pl.ds(h * LN, LN)] + (LOWD + h * A)
    pltpu.async_copy(w1_hbm.at[gidx_v], rows_v, sem).wait()
    for j in range(CPW):
        for c in range(HID // LN):
            s = rows_v[j, pl.ds(c * LN, LN)]
            for h in range(1, H):
                s = s + rows_v[h * LN + j, pl.ds(c * LN, LN)]
            acc_v[j, pl.ds(c * LN, LN)] = s
    pltpu.sync_copy(acc_v, out_hbm.at[pl.ds(base_k, CPW)])


def _fused(act_ref, obs_ref, w_obs_ref, b_obs_ref, w1o_ref, b1c_ref,
           w2T_ref, b2c_ref, idx0_ref, action_ref, value_ref):
    actT = act_ref[...].T
    obs_lowd = jnp.dot(obs_ref[...], w_obs_ref[...],
                       preferred_element_type=jnp.float32) + b_obs_ref[...]
    oT = jax.lax.dot_general(
        w1o_ref[...], obs_lowd, (((0,), (1,)), ((), ())),
        preferred_element_type=jnp.float32) + b1c_ref[...]
    w2T = w2T_ref[...]
    slabs = []
    for b in range(N):
        tb = jnp.maximum(actT + oT[:, b:b + 1], 0.0)
        slabs.append(jax.lax.dot_general(
            w2T, tb, (((1,), (0,)), ((), ())),
            preferred_element_type=jnp.float32))
    out3 = jnp.stack(slabs, axis=1) + b2c_ref[...][:, :, None]
    vals = out3[:H]
    lg = out3[H:]
    m = jnp.max(lg, axis=0, keepdims=True)
    e = jnp.exp(lg - m)
    s = jnp.sum(e, axis=0)
    v = jnp.sum(vals * e, axis=0) / s
    vmax = jnp.max(v, axis=1, keepdims=True)
    iota_k = jax.lax.broadcasted_iota(jnp.int32, (N, K), 1)
    karg = jnp.min(jnp.where(v >= vmax, iota_k, K), axis=1, keepdims=True)
    aidx = jnp.sum(jnp.where(iota_k == karg, idx0_ref[...], 0),
                   axis=1, keepdims=True)
    iota_act = jax.lax.broadcasted_iota(jnp.int32, (N, A), 1)
    action_ref[...] = (iota_act == aidx).astype(jnp.float32)
    value_ref[...] = vmax


@jax.jit
def kernel(observations, action_indices, W_obs, b_obs, W1, b1, W2, b2):
    ai_perm = action_indices.reshape(NW, CPW, H).transpose(0, 2, 1).reshape(K * H)
    act_proj = _sc_act_proj(W1, ai_perm)
    idx0 = action_indices.reshape(K, H)[:, 0].reshape(1, K)
    action, value = pl.pallas_call(
        _fused,
        grid=(1,),
        in_specs=[
            pl.BlockSpec((K, HID), lambda i: (0, 0)),
            pl.BlockSpec((N, OBS_DIM), lambda i: (0, 0)),
            pl.BlockSpec((OBS_DIM, LOWD), lambda i: (0, 0)),
            pl.BlockSpec((1, LOWD), lambda i: (0, 0)),
            pl.BlockSpec((LOWD, HID), lambda i: (0, 0)),
            pl.BlockSpec((HID, 1), lambda i: (0, 0)),
            pl.BlockSpec((2 * H, HID), lambda i: (0, 0)),
            pl.BlockSpec((2 * H, 1), lambda i: (0, 0)),
            pl.BlockSpec((1, K), lambda i: (0, 0)),
        ],
        out_specs=(
            pl.BlockSpec((N, A), lambda i: (0, 0)),
            pl.BlockSpec((N, 1), lambda i: (0, 0)),
        ),
        out_shape=(
            jax.ShapeDtypeStruct((N, A), jnp.float32),
            jax.ShapeDtypeStruct((N, 1), jnp.float32),
        ),
    )(act_proj, observations, W_obs, b_obs.reshape(1, LOWD), W1,
      b1.reshape(HID, 1), W2.T, b2.reshape(2 * H, 1), idx0)
    return (action, value.reshape(N))


# speedup vs baseline: 2.6398x; 2.6398x over previous
import jax
import jax.numpy as jnp
from jax.experimental import pallas as pl

N = 64
OBS_DIM = 1024
LOWD = 512
K = 512
H = 8
A = 128
HID = 512


def _fused(obs_ref, w_obs_ref, b_obs_ref, w1_ref, b1c_ref, idx_ref,
           w2_ref, b2c_ref, action_ref, value_ref):
    idxT_full = idx_ref[...].T
    w2T = w2_ref[...].T
    iota_a = jax.lax.broadcasted_iota(jnp.int32, (A, K), 0)
    actT = jnp.zeros((HID, K), dtype=jnp.float32)
    for h in range(H):
        onehotT = (iota_a == idxT_full[h:h + 1, :]).astype(jnp.float32)
        actT = actT + jax.lax.dot_general(
            w1_ref[LOWD + h * A:LOWD + (h + 1) * A, :], onehotT,
            (((0,), (0,)), ((), ())), preferred_element_type=jnp.float32)

    obs_lowd = jnp.dot(obs_ref[...], w_obs_ref[...],
                       preferred_element_type=jnp.float32) + b_obs_ref[...]
    oT = jax.lax.dot_general(
        w1_ref[:LOWD, :], obs_lowd, (((0,), (1,)), ((), ())),
        preferred_element_type=jnp.float32) + b1c_ref[...]
    slabs = []
    for b in range(N):
        tb = jnp.maximum(actT + oT[:, b:b + 1], 0.0)
        slabs.append(jax.lax.dot_general(
            w2T, tb, (((1,), (0,)), ((), ())),
            preferred_element_type=jnp.float32))
    out3 = jnp.stack(slabs, axis=1) + b2c_ref[...][:, :, None]
    vals = out3[:H]
    lg = out3[H:]
    m = jnp.max(lg, axis=0, keepdims=True)
    e = jnp.exp(lg - m)
    s = jnp.sum(e, axis=0)
    v = jnp.sum(vals * e, axis=0) / s
    vmax = jnp.max(v, axis=1, keepdims=True)
    iota_k = jax.lax.broadcasted_iota(jnp.int32, (N, K), 1)
    karg = jnp.min(jnp.where(v >= vmax, iota_k, K), axis=1, keepdims=True)
    aidx = jnp.sum(jnp.where(iota_k == karg, idxT_full[0:1, :], 0),
                   axis=1, keepdims=True)
    iota_act = jax.lax.broadcasted_iota(jnp.int32, (N, A), 1)
    action_ref[...] = (iota_act == aidx).astype(jnp.float32)
    value_ref[...] = vmax


@jax.jit
def kernel(observations, action_indices, W_obs, b_obs, W1, b1, W2, b2):
    idx = action_indices.reshape(K, H)
    action, value = pl.pallas_call(
        _fused,
        out_shape=(
            jax.ShapeDtypeStruct((N, A), jnp.float32),
            jax.ShapeDtypeStruct((N, 1), jnp.float32),
        ),
    )(observations, W_obs, b_obs.reshape(1, LOWD), W1, b1.reshape(HID, 1),
      idx, W2, b2.reshape(2 * H, 1))
    return (action, value.reshape(N))


# speedup vs baseline: 2.7794x; 1.0529x over previous
import jax
import jax.numpy as jnp
from jax.experimental import pallas as pl

N = 64
OBS_DIM = 1024
LOWD = 512
K = 512
H = 8
A = 128
HID = 512


def _fused(obs_ref, w_obs_ref, b_obs_ref, w1_ref, b1c_ref, idxT_ref,
           w2T_ref, b2c_ref, idx0_ref, action_ref, value_ref):
    iota_a = jax.lax.broadcasted_iota(jnp.int32, (A, K), 0)
    actT = jnp.zeros((HID, K), dtype=jnp.float32)
    for h in range(H):
        onehotT = (iota_a == idxT_ref[h:h + 1, :]).astype(jnp.float32)
        actT = actT + jax.lax.dot_general(
            w1_ref[LOWD + h * A:LOWD + (h + 1) * A, :], onehotT,
            (((0,), (0,)), ((), ())), preferred_element_type=jnp.float32)

    obs_lowd = jnp.dot(obs_ref[...], w_obs_ref[...],
                       preferred_element_type=jnp.float32) + b_obs_ref[...]
    oT = jax.lax.dot_general(
        w1_ref[:LOWD, :], obs_lowd, (((0,), (1,)), ((), ())),
        preferred_element_type=jnp.float32) + b1c_ref[...]
    w2T = w2T_ref[...]
    slabs = []
    for b in range(N):
        tb = jnp.maximum(actT + oT[:, b:b + 1], 0.0)
        slabs.append(jax.lax.dot_general(
            w2T, tb, (((1,), (0,)), ((), ())),
            preferred_element_type=jnp.float32))
    out3 = jnp.stack(slabs, axis=1) + b2c_ref[...][:, :, None]
    vals = out3[:H]
    lg = out3[H:]
    m = jnp.max(lg, axis=0, keepdims=True)
    e = jnp.exp(lg - m)
    s = jnp.sum(e, axis=0)
    v = jnp.sum(vals * e, axis=0) / s
    vmax = jnp.max(v, axis=1, keepdims=True)
    iota_k = jax.lax.broadcasted_iota(jnp.int32, (N, K), 1)
    karg = jnp.min(jnp.where(v >= vmax, iota_k, K), axis=1, keepdims=True)
    aidx = jnp.sum(jnp.where(iota_k == karg, idx0_ref[...], 0),
                   axis=1, keepdims=True)
    iota_act = jax.lax.broadcasted_iota(jnp.int32, (N, A), 1)
    action_ref[...] = (iota_act == aidx).astype(jnp.float32)
    value_ref[...] = vmax


@jax.jit
def kernel(observations, action_indices, W_obs, b_obs, W1, b1, W2, b2):
    idx = action_indices.reshape(K, H)
    action, value = pl.pallas_call(
        _fused,
        out_shape=(
            jax.ShapeDtypeStruct((N, A), jnp.float32),
            jax.ShapeDtypeStruct((N, 1), jnp.float32),
        ),
    )(observations, W_obs, b_obs.reshape(1, LOWD), W1, b1.reshape(HID, 1),
      idx.T, W2.T, b2.reshape(2 * H, 1), idx[:, 0].reshape(1, K))
    return (action, value.reshape(N))
